# baseline (device time: 43884 ns/iter reference)
import functools

import jax
import jax.numpy as jnp
from jax import lax
from jax.experimental import pallas as pl
from jax.experimental.pallas import tpu as pltpu

N_DEV = 8
N_ROUNDS = 3
N_LAYERS = 3
B = 64
D = 1024
H = 2048
MASKS = (1, 3, 4)


def kernel(x, Win0, Wout0, Win1, Wout1, Win2, Wout2):
    def body(
        x_ref,
        win0_ref,
        wout0_ref,
        win1_ref,
        wout1_ref,
        win2_ref,
        wout2_ref,
        out_ref,
        win_stage,
        wout_stage,
        acc_ref,
        send_buf,
        recv_buf,
        rs_send0,
        rs_send1,
        rs_send2,
        rs_recv0,
        rs_recv1,
        rs_recv2,
        load_sems,
        send_sems,
        recv_sems,
    ):
        my = lax.axis_index("i")

        wins = [win0_ref, win1_ref, win2_ref]
        wouts = [wout0_ref, wout1_ref, wout2_ref]

        def stage(l, slot):
            cp_in = pltpu.make_async_copy(
                wins[l], win_stage.at[slot], load_sems.at[slot, 0]
            )
            cp_out = pltpu.make_async_copy(
                wouts[l], wout_stage.at[slot], load_sems.at[slot, 1]
            )
            cp_in.start()
            cp_out.start()
            return cp_in, cp_out

        pending = stage(0, 0)

        barrier = pltpu.get_barrier_semaphore()
        for m in MASKS:
            pl.semaphore_signal(
                barrier,
                inc=1,
                device_id=(my ^ m,),
                device_id_type=pl.DeviceIdType.MESH,
            )
        pl.semaphore_wait(barrier, N_ROUNDS)

        x_bf = x_ref[:, :].astype(jnp.bfloat16)
        cp_in, cp_out = pending
        cp_in.wait()
        win_bf = win_stage[0, :, :].astype(jnp.bfloat16)
        cp_out.wait()
        wout_bf = wout_stage[0, :, :].astype(jnp.bfloat16)

        acc = None
        for l in range(N_LAYERS):
            slot = l % 2
            h = jnp.maximum(
                jnp.dot(x_bf, win_bf, preferred_element_type=jnp.float32),
                0.0,
            ).astype(jnp.bfloat16)
            acc = jnp.dot(h, wout_bf, preferred_element_type=jnp.float32)
            if l + 1 < N_LAYERS:
                cp_in, cp_out = stage(l + 1, 1 - slot)
                for r in range(N_ROUNDS):
                    idx = l * N_ROUNDS + r
                    partner = my ^ MASKS[r]
                    send_buf[idx, :, :] = acc.astype(jnp.bfloat16)
                    rdma = pltpu.make_async_remote_copy(
                        src_ref=send_buf.at[idx],
                        dst_ref=recv_buf.at[idx],
                        send_sem=send_sems.at[idx],
                        recv_sem=recv_sems.at[idx],
                        device_id=(partner,),
                        device_id_type=pl.DeviceIdType.MESH,
                    )
                    rdma.start()
                    if r == 0:
                        cp_in.wait()
                        win_bf = win_stage[1 - slot, :, :].astype(jnp.bfloat16)
                    elif r == 1:
                        cp_out.wait()
                        wout_bf = wout_stage[1 - slot, :, :].astype(
                            jnp.bfloat16
                        )
                    rdma.wait()
                    acc = acc + recv_buf[idx, :, :].astype(jnp.float32)
                x_bf = acc.astype(jnp.bfloat16)

        acc_ref[:, :] = acc

        rs_bufs = [(rs_send0, rs_recv0), (rs_send1, rs_recv1), (rs_send2, rs_recv2)]
        rs_rounds = [
            (4, 32 * (my // 4), 32 * ((my // 4) ^ 1), 32),
            (3, 16 * (my // 2), 16 * ((my ^ 3) // 2), 16),
            (1, 8 * my, 8 * (my ^ 1), 8),
        ]
        for r, (m, keep_off, send_off, nrows) in enumerate(rs_rounds):
            idx = 2 * N_ROUNDS + r
            partner = my ^ m
            sbuf, rbuf = rs_bufs[r]
            sbuf[:, :] = acc_ref[pl.ds(send_off, nrows), :].astype(jnp.bfloat16)
            rdma = pltpu.make_async_remote_copy(
                src_ref=sbuf,
                dst_ref=rbuf,
                send_sem=send_sems.at[idx],
                recv_sem=recv_sems.at[idx],
                device_id=(partner,),
                device_id_type=pl.DeviceIdType.MESH,
            )
            rdma.start()
            rdma.wait()
            acc_ref[pl.ds(keep_off, nrows), :] = acc_ref[
                pl.ds(keep_off, nrows), :
            ] + rbuf[:, :].astype(jnp.float32)

        rows = B // N_DEV
        out_ref[:, :] = acc_ref[pl.ds(my * rows, rows), :]

        @functools.partial(pl.run_scoped, exit_sem=pltpu.SemaphoreType.REGULAR)
        def _(exit_sem):
            for m in MASKS:
                pl.semaphore_signal(
                    exit_sem,
                    inc=1,
                    device_id=(my ^ m,),
                    device_id_type=pl.DeviceIdType.MESH,
                )
            pl.semaphore_wait(exit_sem, N_ROUNDS)

    hbm = pl.BlockSpec(memory_space=pltpu.MemorySpace.HBM)
    vmem = pl.BlockSpec(memory_space=pltpu.VMEM)
    n_ex = N_LAYERS * N_ROUNDS
    return pl.pallas_call(
        body,
        out_shape=jax.ShapeDtypeStruct((B // N_DEV, D), jnp.float32),
        in_specs=[vmem, hbm, hbm, hbm, hbm, hbm, hbm],
        out_specs=vmem,
        scratch_shapes=[
            pltpu.VMEM((2, D, H), jnp.float32),
            pltpu.VMEM((2, H, D), jnp.float32),
            pltpu.VMEM((B, D), jnp.float32),
            pltpu.VMEM((6, B, D), jnp.bfloat16),
            pltpu.VMEM((6, B, D), jnp.bfloat16),
            pltpu.VMEM((B // 2, D), jnp.bfloat16),
            pltpu.VMEM((B // 4, D), jnp.bfloat16),
            pltpu.VMEM((B // 8, D), jnp.bfloat16),
            pltpu.VMEM((B // 2, D), jnp.bfloat16),
            pltpu.VMEM((B // 4, D), jnp.bfloat16),
            pltpu.VMEM((B // 8, D), jnp.bfloat16),
            pltpu.SemaphoreType.DMA((2, 2)),
            pltpu.SemaphoreType.DMA((n_ex,)),
            pltpu.SemaphoreType.DMA((n_ex,)),
        ],
        compiler_params=pltpu.CompilerParams(
            collective_id=0, vmem_limit_bytes=56 * 1024 * 1024
        ),
    )(x, Win0, Wout0, Win1, Wout1, Win2, Wout2)


# device time: 38946 ns/iter; 1.1268x vs baseline; 1.1268x over previous
import functools

import jax
import jax.numpy as jnp
from jax import lax
from jax.experimental import pallas as pl
from jax.experimental.pallas import tpu as pltpu

N_DEV = 8
N_ROUNDS = 3
N_LAYERS = 3
B = 64
D = 1024
H = 2048
MASKS = (1, 3, 4)


def kernel(x, Win0, Wout0, Win1, Wout1, Win2, Wout2):
    def body(
        x_ref,
        win0_ref,
        wout0_ref,
        win1_ref,
        wout1_ref,
        win2_ref,
        wout2_ref,
        out_ref,
        win_stage,
        wout_stage,
        acc_ref,
        send_buf,
        recv_buf,
        rs_send0,
        rs_send1,
        rs_send2,
        rs_recv0,
        rs_recv1,
        rs_recv2,
        load_sems,
        send_sems,
        recv_sems,
        rs_send_sems,
        rs_recv_sems,
    ):
        my = lax.axis_index("i")

        wins = [win0_ref, win1_ref, win2_ref]
        wouts = [wout0_ref, wout1_ref, wout2_ref]

        def stage(l, slot):
            cp_in = pltpu.make_async_copy(
                wins[l], win_stage.at[slot], load_sems.at[slot, 0]
            )
            cp_out = pltpu.make_async_copy(
                wouts[l], wout_stage.at[slot], load_sems.at[slot, 1]
            )
            cp_in.start()
            cp_out.start()
            return cp_in, cp_out

        pending = stage(0, 0)

        barrier = pltpu.get_barrier_semaphore()
        for m in MASKS:
            pl.semaphore_signal(
                barrier,
                inc=1,
                device_id=(my ^ m,),
                device_id_type=pl.DeviceIdType.MESH,
            )
        pl.semaphore_wait(barrier, N_ROUNDS)

        HALF = D // 2

        def half_exchange(l, r, half, partner, val):
            idx = (l * N_ROUNDS + r) * 2 + half
            send_buf[idx, :, :] = val.astype(jnp.bfloat16)
            rdma = pltpu.make_async_remote_copy(
                src_ref=send_buf.at[idx],
                dst_ref=recv_buf.at[idx],
                send_sem=send_sems.at[idx],
                recv_sem=recv_sems.at[idx],
                device_id=(partner,),
                device_id_type=pl.DeviceIdType.MESH,
            )
            rdma.start()
            return rdma, idx

        x_bf = x_ref[:, :].astype(jnp.bfloat16)
        acc = None
        for l in range(N_LAYERS):
            slot = l % 2
            cp_in, cp_out = pending
            cp_in.wait()
            h = jnp.maximum(
                jnp.dot(
                    x_bf,
                    win_stage[slot, :, :].astype(jnp.bfloat16),
                    preferred_element_type=jnp.float32,
                ),
                0.0,
            ).astype(jnp.bfloat16)
            cp_out.wait()
            acc = jnp.dot(
                h,
                wout_stage[slot, :, :].astype(jnp.bfloat16),
                preferred_element_type=jnp.float32,
            )
            if l + 1 < N_LAYERS:
                pending = stage(l + 1, 1 - slot)
                ha = acc[:, :HALF]
                hb = acc[:, HALF:]
                rd_a, ia = half_exchange(l, 0, 0, my ^ MASKS[0], ha)
                rd_b, ib = half_exchange(l, 0, 1, my ^ MASKS[0], hb)
                for r in range(N_ROUNDS):
                    rd_a.wait()
                    ha = ha + recv_buf[ia, :, :].astype(jnp.float32)
                    if r + 1 < N_ROUNDS:
                        rd_a, ia = half_exchange(
                            l, r + 1, 0, my ^ MASKS[r + 1], ha
                        )
                    rd_b.wait()
                    hb = hb + recv_buf[ib, :, :].astype(jnp.float32)
                    if r + 1 < N_ROUNDS:
                        rd_b, ib = half_exchange(
                            l, r + 1, 1, my ^ MASKS[r + 1], hb
                        )
                acc = jnp.concatenate([ha, hb], axis=1)
                x_bf = acc.astype(jnp.bfloat16)

        acc_ref[:, :] = acc

        rs_bufs = [(rs_send0, rs_recv0), (rs_send1, rs_recv1), (rs_send2, rs_recv2)]
        rs_rounds = [
            (4, 32 * (my // 4), 32 * ((my // 4) ^ 1), 32),
            (3, 16 * (my // 2), 16 * ((my ^ 3) // 2), 16),
            (1, 8 * my, 8 * (my ^ 1), 8),
        ]
        for r, (m, keep_off, send_off, nrows) in enumerate(rs_rounds):
            partner = my ^ m
            sbuf, rbuf = rs_bufs[r]
            sbuf[:, :] = acc_ref[pl.ds(send_off, nrows), :].astype(jnp.bfloat16)
            rdma = pltpu.make_async_remote_copy(
                src_ref=sbuf,
                dst_ref=rbuf,
                send_sem=rs_send_sems.at[r],
                recv_sem=rs_recv_sems.at[r],
                device_id=(partner,),
                device_id_type=pl.DeviceIdType.MESH,
            )
            rdma.start()
            rdma.wait()
            acc_ref[pl.ds(keep_off, nrows), :] = acc_ref[
                pl.ds(keep_off, nrows), :
            ] + rbuf[:, :].astype(jnp.float32)

        rows = B // N_DEV
        out_ref[:, :] = acc_ref[pl.ds(my * rows, rows), :]

        @functools.partial(pl.run_scoped, exit_sem=pltpu.SemaphoreType.REGULAR)
        def _(exit_sem):
            for m in MASKS:
                pl.semaphore_signal(
                    exit_sem,
                    inc=1,
                    device_id=(my ^ m,),
                    device_id_type=pl.DeviceIdType.MESH,
                )
            pl.semaphore_wait(exit_sem, N_ROUNDS)

    hbm = pl.BlockSpec(memory_space=pltpu.MemorySpace.HBM)
    vmem = pl.BlockSpec(memory_space=pltpu.VMEM)
    n_ex = N_LAYERS * N_ROUNDS
    return pl.pallas_call(
        body,
        out_shape=jax.ShapeDtypeStruct((B // N_DEV, D), jnp.float32),
        in_specs=[vmem, hbm, hbm, hbm, hbm, hbm, hbm],
        out_specs=vmem,
        scratch_shapes=[
            pltpu.VMEM((2, D, H), jnp.float32),
            pltpu.VMEM((2, H, D), jnp.float32),
            pltpu.VMEM((B, D), jnp.float32),
            pltpu.VMEM((12, B, D // 2), jnp.bfloat16),
            pltpu.VMEM((12, B, D // 2), jnp.bfloat16),
            pltpu.VMEM((B // 2, D), jnp.bfloat16),
            pltpu.VMEM((B // 4, D), jnp.bfloat16),
            pltpu.VMEM((B // 8, D), jnp.bfloat16),
            pltpu.VMEM((B // 2, D), jnp.bfloat16),
            pltpu.VMEM((B // 4, D), jnp.bfloat16),
            pltpu.VMEM((B // 8, D), jnp.bfloat16),
            pltpu.SemaphoreType.DMA((2, 2)),
            pltpu.SemaphoreType.DMA((12,)),
            pltpu.SemaphoreType.DMA((12,)),
            pltpu.SemaphoreType.DMA((3,)),
            pltpu.SemaphoreType.DMA((3,)),
        ],
        compiler_params=pltpu.CompilerParams(
            collective_id=0, vmem_limit_bytes=56 * 1024 * 1024
        ),
    )(x, Win0, Wout0, Win1, Wout1, Win2, Wout2)


# device time: 38665 ns/iter; 1.1350x vs baseline; 1.0073x over previous
import functools

import jax
import jax.numpy as jnp
from jax import lax
from jax.experimental import pallas as pl
from jax.experimental.pallas import tpu as pltpu

N_DEV = 8
N_ROUNDS = 3
N_LAYERS = 3
B = 64
D = 1024
H = 2048
MASKS = (1, 3, 4)


def kernel(x, Win0, Wout0, Win1, Wout1, Win2, Wout2):
    def body(
        x_ref,
        win0_ref,
        wout0_ref,
        win1_ref,
        wout1_ref,
        win2_ref,
        wout2_ref,
        out_ref,
        win_stage,
        wout_stage,
        acc_ref,
        send_buf,
        recv_buf,
        rs_send0,
        rs_send1,
        rs_send2,
        rs_recv0,
        rs_recv1,
        rs_recv2,
        load_sems,
        send_sems,
        recv_sems,
        rs_send_sems,
        rs_recv_sems,
    ):
        my = lax.axis_index("i")

        wins = [win0_ref, win1_ref, win2_ref]
        wouts = [wout0_ref, wout1_ref, wout2_ref]

        def stage(l, slot):
            cp_in = pltpu.make_async_copy(
                wins[l], win_stage.at[slot], load_sems.at[slot, 0]
            )
            cp_out = pltpu.make_async_copy(
                wouts[l], wout_stage.at[slot], load_sems.at[slot, 1]
            )
            cp_in.start()
            cp_out.start()
            return cp_in, cp_out

        pending = stage(0, 0)

        barrier = pltpu.get_barrier_semaphore()
        for m in MASKS:
            pl.semaphore_signal(
                barrier,
                inc=1,
                device_id=(my ^ m,),
                device_id_type=pl.DeviceIdType.MESH,
            )
        pl.semaphore_wait(barrier, N_ROUNDS)

        HALF = D // 2

        def half_exchange(l, r, half, partner, val):
            idx = (l * N_ROUNDS + r) * 2 + half
            send_buf[idx, :, :] = val.astype(jnp.bfloat16)
            rdma = pltpu.make_async_remote_copy(
                src_ref=send_buf.at[idx],
                dst_ref=recv_buf.at[idx],
                send_sem=send_sems.at[idx],
                recv_sem=recv_sems.at[idx],
                device_id=(partner,),
                device_id_type=pl.DeviceIdType.MESH,
            )
            rdma.start()
            return rdma, idx

        x_bf = x_ref[:, :].astype(jnp.bfloat16)
        cp_in, cp_out = pending
        cp_in.wait()
        win_bf = win_stage[0, :, :].astype(jnp.bfloat16)
        acc = None
        for l in range(N_LAYERS):
            slot = l % 2
            h = jnp.maximum(
                jnp.dot(x_bf, win_bf, preferred_element_type=jnp.float32),
                0.0,
            ).astype(jnp.bfloat16)
            if l == 0:
                cp_out.wait()
                wout_bf = wout_stage[0, :, :].astype(jnp.bfloat16)
            acc = jnp.dot(h, wout_bf, preferred_element_type=jnp.float32)
            if l + 1 < N_LAYERS:
                cp_in, cp_out = stage(l + 1, 1 - slot)
                ha = acc[:, :HALF]
                hb = acc[:, HALF:]
                rd_a, ia = half_exchange(l, 0, 0, my ^ MASKS[0], ha)
                rd_b, ib = half_exchange(l, 0, 1, my ^ MASKS[0], hb)
                for r in range(N_ROUNDS):
                    if r == 1:
                        cp_in.wait()
                        win_bf = win_stage[1 - slot, :, :].astype(jnp.bfloat16)
                    elif r == 2:
                        cp_out.wait()
                        wout_bf = wout_stage[1 - slot, :, :].astype(
                            jnp.bfloat16
                        )
                    rd_a.wait()
                    ha = ha + recv_buf[ia, :, :].astype(jnp.float32)
                    if r + 1 < N_ROUNDS:
                        rd_a, ia = half_exchange(
                            l, r + 1, 0, my ^ MASKS[r + 1], ha
                        )
                    rd_b.wait()
                    hb = hb + recv_buf[ib, :, :].astype(jnp.float32)
                    if r + 1 < N_ROUNDS:
                        rd_b, ib = half_exchange(
                            l, r + 1, 1, my ^ MASKS[r + 1], hb
                        )
                acc = jnp.concatenate([ha, hb], axis=1)
                x_bf = acc.astype(jnp.bfloat16)

        acc_ref[:, :] = acc

        rs_bufs = [(rs_send0, rs_recv0), (rs_send1, rs_recv1), (rs_send2, rs_recv2)]
        rs_rounds = [
            (4, 32 * (my // 4), 32 * ((my // 4) ^ 1), 32),
            (3, 16 * (my // 2), 16 * ((my ^ 3) // 2), 16),
            (1, 8 * my, 8 * (my ^ 1), 8),
        ]
        for r, (m, keep_off, send_off, nrows) in enumerate(rs_rounds):
            partner = my ^ m
            sbuf, rbuf = rs_bufs[r]
            sbuf[:, :] = acc_ref[pl.ds(send_off, nrows), :].astype(jnp.bfloat16)
            rdma = pltpu.make_async_remote_copy(
                src_ref=sbuf,
                dst_ref=rbuf,
                send_sem=rs_send_sems.at[r],
                recv_sem=rs_recv_sems.at[r],
                device_id=(partner,),
                device_id_type=pl.DeviceIdType.MESH,
            )
            rdma.start()
            rdma.wait()
            acc_ref[pl.ds(keep_off, nrows), :] = acc_ref[
                pl.ds(keep_off, nrows), :
            ] + rbuf[:, :].astype(jnp.float32)

        rows = B // N_DEV
        out_ref[:, :] = acc_ref[pl.ds(my * rows, rows), :]

        @functools.partial(pl.run_scoped, exit_sem=pltpu.SemaphoreType.REGULAR)
        def _(exit_sem):
            for m in MASKS:
                pl.semaphore_signal(
                    exit_sem,
                    inc=1,
                    device_id=(my ^ m,),
                    device_id_type=pl.DeviceIdType.MESH,
                )
            pl.semaphore_wait(exit_sem, N_ROUNDS)

    hbm = pl.BlockSpec(memory_space=pltpu.MemorySpace.HBM)
    vmem = pl.BlockSpec(memory_space=pltpu.VMEM)
    n_ex = N_LAYERS * N_ROUNDS
    return pl.pallas_call(
        body,
        out_shape=jax.ShapeDtypeStruct((B // N_DEV, D), jnp.float32),
        in_specs=[vmem, hbm, hbm, hbm, hbm, hbm, hbm],
        out_specs=vmem,
        scratch_shapes=[
            pltpu.VMEM((2, D, H), jnp.float32),
            pltpu.VMEM((2, H, D), jnp.float32),
            pltpu.VMEM((B, D), jnp.float32),
            pltpu.VMEM((12, B, D // 2), jnp.bfloat16),
            pltpu.VMEM((12, B, D // 2), jnp.bfloat16),
            pltpu.VMEM((B // 2, D), jnp.bfloat16),
            pltpu.VMEM((B // 4, D), jnp.bfloat16),
            pltpu.VMEM((B // 8, D), jnp.bfloat16),
            pltpu.VMEM((B // 2, D), jnp.bfloat16),
            pltpu.VMEM((B // 4, D), jnp.bfloat16),
            pltpu.VMEM((B // 8, D), jnp.bfloat16),
            pltpu.SemaphoreType.DMA((2, 2)),
            pltpu.SemaphoreType.DMA((12,)),
            pltpu.SemaphoreType.DMA((12,)),
            pltpu.SemaphoreType.DMA((3,)),
            pltpu.SemaphoreType.DMA((3,)),
        ],
        compiler_params=pltpu.CompilerParams(
            collective_id=0, vmem_limit_bytes=56 * 1024 * 1024
        ),
    )(x, Win0, Wout0, Win1, Wout1, Win2, Wout2)


# device time: 36388 ns/iter; 1.2060x vs baseline; 1.0626x over previous
import functools

import jax
import jax.numpy as jnp
from jax import lax
from jax.experimental import pallas as pl
from jax.experimental.pallas import tpu as pltpu

N_DEV = 8
N_ROUNDS = 3
N_LAYERS = 3
B = 64
D = 1024
H = 2048
MASKS = (1, 3, 4)


def kernel(x, Win0, Wout0, Win1, Wout1, Win2, Wout2):
    def body(
        x_ref,
        win0_ref,
        wout0_ref,
        win1_ref,
        wout1_ref,
        win2_ref,
        wout2_ref,
        out_ref,
        win_stage,
        wout_stage,
        acc_ref,
        send_buf,
        recv_buf,
        rs_send,
        rs_recv,
        load_sems,
        send_sems,
        recv_sems,
        rs_send_sems,
        rs_recv_sems,
    ):
        my = lax.axis_index("i")

        wins = [win0_ref, win1_ref, win2_ref]
        wouts = [wout0_ref, wout1_ref, wout2_ref]

        def stage(l, slot):
            cp_in = pltpu.make_async_copy(
                wins[l], win_stage.at[slot], load_sems.at[slot, 0]
            )
            cp_out = pltpu.make_async_copy(
                wouts[l], wout_stage.at[slot], load_sems.at[slot, 1]
            )
            cp_in.start()
            cp_out.start()
            return cp_in, cp_out

        pending = stage(0, 0)

        barrier = pltpu.get_barrier_semaphore()
        for o in range(1, N_DEV):
            pl.semaphore_signal(
                barrier,
                inc=1,
                device_id=(my ^ o,),
                device_id_type=pl.DeviceIdType.MESH,
            )
        pl.semaphore_wait(barrier, N_DEV - 1)

        HALF = D // 2

        def half_exchange(l, r, half, partner, val):
            idx = (l * N_ROUNDS + r) * 2 + half
            send_buf[idx, :, :] = val.astype(jnp.bfloat16)
            rdma = pltpu.make_async_remote_copy(
                src_ref=send_buf.at[idx],
                dst_ref=recv_buf.at[idx],
                send_sem=send_sems.at[idx],
                recv_sem=recv_sems.at[idx],
                device_id=(partner,),
                device_id_type=pl.DeviceIdType.MESH,
            )
            rdma.start()
            return rdma, idx

        x_bf = x_ref[:, :].astype(jnp.bfloat16)
        cp_in, cp_out = pending
        cp_in.wait()
        win_bf = win_stage[0, :, :].astype(jnp.bfloat16)
        acc = None
        for l in range(N_LAYERS):
            slot = l % 2
            h = jnp.maximum(
                jnp.dot(x_bf, win_bf, preferred_element_type=jnp.float32),
                0.0,
            ).astype(jnp.bfloat16)
            if l == 0:
                cp_out.wait()
                wout_bf = wout_stage[0, :, :].astype(jnp.bfloat16)
            acc = jnp.dot(h, wout_bf, preferred_element_type=jnp.float32)
            if l + 1 < N_LAYERS:
                cp_in, cp_out = stage(l + 1, 1 - slot)
                ha = acc[:, :HALF]
                hb = acc[:, HALF:]
                rd_a, ia = half_exchange(l, 0, 0, my ^ MASKS[0], ha)
                rd_b, ib = half_exchange(l, 0, 1, my ^ MASKS[0], hb)
                for r in range(N_ROUNDS):
                    if r == 1:
                        cp_in.wait()
                        win_bf = win_stage[1 - slot, :, :].astype(jnp.bfloat16)
                    elif r == 2:
                        cp_out.wait()
                        wout_bf = wout_stage[1 - slot, :, :].astype(
                            jnp.bfloat16
                        )
                    rd_a.wait()
                    ha = ha + recv_buf[ia, :, :].astype(jnp.float32)
                    if r + 1 < N_ROUNDS:
                        rd_a, ia = half_exchange(
                            l, r + 1, 0, my ^ MASKS[r + 1], ha
                        )
                    rd_b.wait()
                    hb = hb + recv_buf[ib, :, :].astype(jnp.float32)
                    if r + 1 < N_ROUNDS:
                        rd_b, ib = half_exchange(
                            l, r + 1, 1, my ^ MASKS[r + 1], hb
                        )
                acc = jnp.concatenate([ha, hb], axis=1)
                x_bf = acc.astype(jnp.bfloat16)

        acc_ref[:, :] = acc

        rows = B // N_DEV
        rdmas = []
        for o in range(1, N_DEV):
            q = my ^ o
            rs_send[o, :, :] = acc_ref[pl.ds(q * rows, rows), :].astype(
                jnp.bfloat16
            )
            rdma = pltpu.make_async_remote_copy(
                src_ref=rs_send.at[o],
                dst_ref=rs_recv.at[o],
                send_sem=rs_send_sems.at[o],
                recv_sem=rs_recv_sems.at[o],
                device_id=(q,),
                device_id_type=pl.DeviceIdType.MESH,
            )
            rdma.start()
            rdmas.append(rdma)
        mine = acc_ref[pl.ds(my * rows, rows), :]
        for o, rdma in enumerate(rdmas, start=1):
            rdma.wait()
            mine = mine + rs_recv[o, :, :].astype(jnp.float32)
        out_ref[:, :] = mine

        @functools.partial(pl.run_scoped, exit_sem=pltpu.SemaphoreType.REGULAR)
        def _(exit_sem):
            for o in range(1, N_DEV):
                pl.semaphore_signal(
                    exit_sem,
                    inc=1,
                    device_id=(my ^ o,),
                    device_id_type=pl.DeviceIdType.MESH,
                )
            pl.semaphore_wait(exit_sem, N_DEV - 1)

    hbm = pl.BlockSpec(memory_space=pltpu.MemorySpace.HBM)
    vmem = pl.BlockSpec(memory_space=pltpu.VMEM)
    n_ex = N_LAYERS * N_ROUNDS
    return pl.pallas_call(
        body,
        out_shape=jax.ShapeDtypeStruct((B // N_DEV, D), jnp.float32),
        in_specs=[vmem, hbm, hbm, hbm, hbm, hbm, hbm],
        out_specs=vmem,
        scratch_shapes=[
            pltpu.VMEM((2, D, H), jnp.float32),
            pltpu.VMEM((2, H, D), jnp.float32),
            pltpu.VMEM((B, D), jnp.float32),
            pltpu.VMEM((12, B, D // 2), jnp.bfloat16),
            pltpu.VMEM((12, B, D // 2), jnp.bfloat16),
            pltpu.VMEM((N_DEV, B // N_DEV, D), jnp.bfloat16),
            pltpu.VMEM((N_DEV, B // N_DEV, D), jnp.bfloat16),
            pltpu.SemaphoreType.DMA((2, 2)),
            pltpu.SemaphoreType.DMA((12,)),
            pltpu.SemaphoreType.DMA((12,)),
            pltpu.SemaphoreType.DMA((N_DEV,)),
            pltpu.SemaphoreType.DMA((N_DEV,)),
        ],
        compiler_params=pltpu.CompilerParams(
            collective_id=0, vmem_limit_bytes=56 * 1024 * 1024
        ),
    )(x, Win0, Wout0, Win1, Wout1, Win2, Wout2)
